# baseline (speedup 1.0000x reference)
import functools

import jax
import jax.numpy as jnp
from jax import lax
from jax.experimental import pallas as pl
from jax.experimental.pallas import tpu as pltpu
from jax.experimental.pallas import tpu_sc as plsc

N = 10000
E = 320000
F = 128
NC = 2
NS = 16
NW = NC * NS
BLK = 128
NB = 80
EPAD = NW * NB * BLK
FQ = F // 4
NBT = EPAD // (NS * BLK)
NPAD = 10112
ROWS_T = 640
ROWS_L = NPAD - (NS - 1) * ROWS_T


def _sc_mesh():
    return plsc.VectorSubcoreMesh(core_axis_name="c", subcore_axis_name="s")


_RSQRT_MAGIC = 0x5F3759DF


def _newton_rsqrt(d):
    i = lax.bitcast_convert_type(d, jnp.int32)
    i = _RSQRT_MAGIC - lax.shift_right_arithmetic(i, 1)
    y = lax.bitcast_convert_type(i, jnp.float32)
    h = 0.5 * d
    for _ in range(3):
        y = y * (1.5 - h * y * y)
    return jnp.where(d >= 0.5, y, 0.0)


@functools.partial(
    pl.kernel,
    out_type=[
        jax.ShapeDtypeStruct((4, NPAD, FQ), jnp.float32),
        jax.ShapeDtypeStruct((NC, NPAD), jnp.float32),
    ],
    mesh=_sc_mesh(),
    scratch_types=[
        pltpu.VMEM((NBT, BLK), jnp.int32),
        pltpu.VMEM((NBT, BLK), jnp.int32),
        pltpu.VMEM((BLK, FQ), jnp.float32),
        pltpu.VMEM((BLK, FQ), jnp.float32),
        pltpu.VMEM((BLK, FQ), jnp.float32),
        pltpu.VMEM((BLK,), jnp.float32),
        pltpu.VMEM((ROWS_T,), jnp.float32),
        pltpu.VMEM_SHARED((NPAD, FQ), jnp.float32),
        pltpu.VMEM_SHARED((NPAD, FQ), jnp.float32),
        pltpu.VMEM_SHARED((NPAD,), jnp.float32),
        pltpu.SemaphoreType.DMA,
        pltpu.SemaphoreType.DMA,
        pltpu.SemaphoreType.DMA,
        pltpu.SemaphoreType.DMA,
        pltpu.SemaphoreType.DMA,
    ],
    compiler_params=pltpu.CompilerParams(use_tc_tiling_on_sc=False),
)
def _spmm1_kernel(x4_hbm, src_hbm, dst_hbm, zrows_hbm, ones_hbm, zer1_hbm,
                  out_hbm, dis_hbm,
                  sidx, didx, rows0, rows1, zrows, onesv, disv,
                  tab, acc, dega, sem0, sem1, sem2, sem0s, sem1s):
    c = lax.axis_index("c")
    s = lax.axis_index("s")
    r0 = s * ROWS_T
    nr = jnp.where(s == NS - 1, ROWS_L, ROWS_T)
    nz = nr // BLK
    nd = jnp.where(s == NS - 1, (N - (NS - 1) * ROWS_T) // 80, ROWS_T // 80)

    pltpu.sync_copy(src_hbm.at[s], sidx)
    pltpu.sync_copy(dst_hbm.at[s], didx)
    pltpu.sync_copy(zrows_hbm, zrows)
    pltpu.sync_copy(ones_hbm, onesv)

    pltpu.sync_copy(zer1_hbm, disv)

    @pl.when(s < NS - 1)
    def _():
        pltpu.sync_copy(disv, dega.at[pl.ds(r0, ROWS_T)])

    @pl.when(s == NS - 1)
    def _():
        pltpu.sync_copy(disv.at[pl.ds(0, ROWS_L)],
                        dega.at[pl.ds(r0, ROWS_L)])

    plsc.subcore_barrier()

    def dfire(j, carry):
        pltpu.async_copy(onesv, dega.at[sidx.at[j]], sem2, add=True)
        return carry

    lax.fori_loop(0, NBT, dfire, 0)

    def ddrain(j, carry):
        pltpu.make_async_copy(onesv, dega.at[sidx.at[0]], sem2).wait()
        return carry

    lax.fori_loop(0, NBT, ddrain, 0)
    plsc.subcore_barrier()

    @pl.when(s < NS - 1)
    def _():
        pltpu.sync_copy(dega.at[pl.ds(r0, ROWS_T)], disv)

    @pl.when(s == NS - 1)
    def _():
        pltpu.sync_copy(dega.at[pl.ds(r0, ROWS_L)], disv.at[pl.ds(0, ROWS_L)])

    def disbody(k, carry):
        d = disv[pl.ds(16 * k, 16)]
        disv[pl.ds(16 * k, 16)] = _newton_rsqrt(d)
        return carry

    lax.fori_loop(0, nr // 16, disbody, 0)

    @pl.when(s < NS - 1)
    def _():
        pltpu.async_copy(disv, dis_hbm.at[c, pl.ds(r0, ROWS_T)], sem2)

    @pl.when(s == NS - 1)
    def _():
        pltpu.async_copy(disv.at[pl.ds(0, ROWS_L)],
                         dis_hbm.at[c, pl.ds(r0, ROWS_L)], sem2)

    for q in range(2):
        fq = 2 * c + q

        pltpu.async_copy(x4_hbm.at[fq, pl.ds(r0, BLK)], rows0, sem0)
        for k in range(5):
            buf = rows0 if k % 2 == 0 else rows1
            sem = sem0 if k % 2 == 0 else sem1
            nbuf = rows1 if k % 2 == 0 else rows0
            nsem = sem1 if k % 2 == 0 else sem0

            @pl.when(k < nz)
            def _(k=k, buf=buf, sem=sem, nbuf=nbuf, nsem=nsem):
                pltpu.make_async_copy(x4_hbm.at[fq, pl.ds(r0, BLK)], buf,
                                      sem).wait()

                @pl.when(k + 1 < nz)
                def _():
                    pltpu.async_copy(
                        x4_hbm.at[fq, pl.ds(r0 + (k + 1) * BLK, BLK)],
                        nbuf, nsem)

                def srow(r, carry, k=k, buf=buf):
                    d = disv[pl.ds(k * BLK + r, 1)][0]
                    buf[r, pl.ds(0, 16)] = buf[r, pl.ds(0, 16)] * d
                    buf[r, pl.ds(16, 16)] = buf[r, pl.ds(16, 16)] * d
                    return carry

                lax.fori_loop(0, BLK, srow, 0)
                pltpu.sync_copy(buf, tab.at[pl.ds(r0 + k * BLK, BLK)])
                pltpu.sync_copy(zrows, acc.at[pl.ds(r0 + k * BLK, BLK)])

        plsc.subcore_barrier()

        pltpu.async_copy(tab.at[sidx.at[0]], rows0, sem0)
        pltpu.async_copy(tab.at[sidx.at[1]], rows1, sem1)

        def body(i, carry):
            t0 = 2 * i
            pltpu.make_async_copy(tab.at[sidx.at[t0]], rows0, sem0).wait()
            pltpu.async_copy(rows0, acc.at[didx.at[t0]], sem0s, add=True)
            pltpu.make_async_copy(tab.at[sidx.at[t0 + 1]], rows1, sem1).wait()
            pltpu.async_copy(rows1, acc.at[didx.at[t0 + 1]], sem1s, add=True)

            @pl.when(t0 + 2 < NBT)
            def _():
                pltpu.make_async_copy(rows0, acc.at[didx.at[t0]], sem0s).wait()
                pltpu.async_copy(tab.at[sidx.at[t0 + 2]], rows0, sem0)

            @pl.when(t0 + 3 < NBT)
            def _():
                pltpu.make_async_copy(rows1, acc.at[didx.at[t0 + 1]],
                                      sem1s).wait()
                pltpu.async_copy(tab.at[sidx.at[t0 + 3]], rows1, sem1)

            return carry

        lax.fori_loop(0, NBT // 2, body, 0)
        pltpu.make_async_copy(rows0, acc.at[didx.at[0]], sem0s).wait()
        pltpu.make_async_copy(rows1, acc.at[didx.at[0]], sem1s).wait()
        plsc.subcore_barrier()

        for k in range(ROWS_T // 80):
            buf = rows0 if k % 2 == 0 else rows1
            sem = sem0 if k % 2 == 0 else sem1

            @pl.when(k < nd)
            def _(k=k, buf=buf, sem=sem):
                if k >= 2:
                    pltpu.make_async_copy(buf.at[pl.ds(0, 80)],
                                          out_hbm.at[fq, pl.ds(r0, 80)],
                                          sem).wait()
                pltpu.sync_copy(acc.at[pl.ds(r0 + k * 80, 80)],
                                buf.at[pl.ds(0, 80)])
                pltpu.async_copy(buf.at[pl.ds(0, 80)],
                                 out_hbm.at[fq, pl.ds(r0 + k * 80, 80)], sem)

        for k in range(2):
            buf = rows0 if k == 0 else rows1
            sem = sem0 if k == 0 else sem1

            @pl.when(k < nd)
            def _(buf=buf, sem=sem):
                pltpu.make_async_copy(buf.at[pl.ds(0, 80)],
                                      out_hbm.at[fq, pl.ds(r0, 80)], sem).wait()

        plsc.subcore_barrier()

    @pl.when(s < NS - 1)
    def _():
        pltpu.make_async_copy(disv, dis_hbm.at[c, pl.ds(r0, ROWS_T)],
                              sem2).wait()

    @pl.when(s == NS - 1)
    def _():
        pltpu.make_async_copy(disv.at[pl.ds(0, ROWS_L)],
                              dis_hbm.at[c, pl.ds(r0, ROWS_L)], sem2).wait()


@functools.partial(
    pl.kernel,
    out_type=jax.ShapeDtypeStruct((4, NPAD, FQ), jnp.float32),
    mesh=_sc_mesh(),
    scratch_types=[
        pltpu.VMEM((NBT, BLK), jnp.int32),
        pltpu.VMEM((NBT, BLK), jnp.int32),
        pltpu.VMEM((BLK, FQ), jnp.float32),
        pltpu.VMEM((BLK, FQ), jnp.float32),
        pltpu.VMEM((BLK, FQ), jnp.float32),
        pltpu.VMEM_SHARED((NPAD, FQ), jnp.float32),
        pltpu.VMEM_SHARED((NPAD, FQ), jnp.float32),
        pltpu.SemaphoreType.DMA,
        pltpu.SemaphoreType.DMA,
        pltpu.SemaphoreType.DMA,
        pltpu.SemaphoreType.DMA,
    ],
    compiler_params=pltpu.CompilerParams(use_tc_tiling_on_sc=False),
)
def _spmm_kernel(tabs_hbm, src_hbm, dst_hbm, zrows_hbm, out_hbm,
                 sidx, didx, rows0, rows1, zrows, tab, acc,
                 sem0, sem1, sem0s, sem1s):
    c = lax.axis_index("c")
    s = lax.axis_index("s")
    r0 = s * ROWS_T
    nr = jnp.where(s == NS - 1, ROWS_L, ROWS_T)
    nz = nr // BLK
    nd = jnp.where(s == NS - 1, (N - (NS - 1) * ROWS_T) // 80, ROWS_T // 80)

    pltpu.sync_copy(src_hbm.at[s], sidx)
    pltpu.sync_copy(dst_hbm.at[s], didx)
    pltpu.sync_copy(zrows_hbm, zrows)

    for q in range(2):
        fq = 2 * c + q

        pltpu.async_copy(tabs_hbm.at[fq, pl.ds(r0, BLK)], rows0, sem0)
        for k in range(5):
            buf = rows0 if k % 2 == 0 else rows1
            sem = sem0 if k % 2 == 0 else sem1
            nbuf = rows1 if k % 2 == 0 else rows0
            nsem = sem1 if k % 2 == 0 else sem0

            @pl.when(k < nz)
            def _(k=k, buf=buf, sem=sem, nbuf=nbuf, nsem=nsem):
                pltpu.make_async_copy(tabs_hbm.at[fq, pl.ds(r0, BLK)], buf,
                                      sem).wait()

                @pl.when(k + 1 < nz)
                def _():
                    pltpu.async_copy(
                        tabs_hbm.at[fq, pl.ds(r0 + (k + 1) * BLK, BLK)],
                        nbuf, nsem)

                pltpu.sync_copy(buf, tab.at[pl.ds(r0 + k * BLK, BLK)])
                pltpu.sync_copy(zrows, acc.at[pl.ds(r0 + k * BLK, BLK)])

        plsc.subcore_barrier()

        pltpu.async_copy(tab.at[sidx.at[0]], rows0, sem0)
        pltpu.async_copy(tab.at[sidx.at[1]], rows1, sem1)

        def body(i, carry):
            t0 = 2 * i
            pltpu.make_async_copy(tab.at[sidx.at[t0]], rows0, sem0).wait()
            pltpu.async_copy(rows0, acc.at[didx.at[t0]], sem0s, add=True)
            pltpu.make_async_copy(tab.at[sidx.at[t0 + 1]], rows1, sem1).wait()
            pltpu.async_copy(rows1, acc.at[didx.at[t0 + 1]], sem1s, add=True)

            @pl.when(t0 + 2 < NBT)
            def _():
                pltpu.make_async_copy(rows0, acc.at[didx.at[t0]], sem0s).wait()
                pltpu.async_copy(tab.at[sidx.at[t0 + 2]], rows0, sem0)

            @pl.when(t0 + 3 < NBT)
            def _():
                pltpu.make_async_copy(rows1, acc.at[didx.at[t0 + 1]],
                                      sem1s).wait()
                pltpu.async_copy(tab.at[sidx.at[t0 + 3]], rows1, sem1)

            return carry

        lax.fori_loop(0, NBT // 2, body, 0)
        pltpu.make_async_copy(rows0, acc.at[didx.at[0]], sem0s).wait()
        pltpu.make_async_copy(rows1, acc.at[didx.at[0]], sem1s).wait()
        plsc.subcore_barrier()

        for k in range(ROWS_T // 80):
            buf = rows0 if k % 2 == 0 else rows1
            sem = sem0 if k % 2 == 0 else sem1

            @pl.when(k < nd)
            def _(k=k, buf=buf, sem=sem):
                if k >= 2:
                    pltpu.make_async_copy(buf.at[pl.ds(0, 80)],
                                          out_hbm.at[fq, pl.ds(r0, 80)],
                                          sem).wait()
                pltpu.sync_copy(acc.at[pl.ds(r0 + k * 80, 80)],
                                buf.at[pl.ds(0, 80)])
                pltpu.async_copy(buf.at[pl.ds(0, 80)],
                                 out_hbm.at[fq, pl.ds(r0 + k * 80, 80)], sem)

        for k in range(2):
            buf = rows0 if k == 0 else rows1
            sem = sem0 if k == 0 else sem1

            @pl.when(k < nd)
            def _(buf=buf, sem=sem):
                pltpu.make_async_copy(buf.at[pl.ds(0, 80)],
                                      out_hbm.at[fq, pl.ds(r0, 80)], sem).wait()

        plsc.subcore_barrier()


def _mid_body(x_ref, s_ref, dis_ref, w0_ref, w1_ref, b_ref, h_ref, hs4_ref):
    ndis = -dis_ref[...]
    t = jnp.concatenate([s_ref[k] for k in range(4)], axis=1) * ndis
    h = (jnp.dot(x_ref[...], w0_ref[...], preferred_element_type=jnp.float32)
         + jnp.dot(t, w1_ref[...], preferred_element_type=jnp.float32)
         + b_ref[...])
    h = jnp.maximum(h, 0.0)
    h_ref[...] = h
    hs = h * dis_ref[...]
    for k in range(4):
        hs4_ref[k] = hs[:, k * FQ:(k + 1) * FQ]


def _mid_call(x, S, dis, W0, W1, b):
    blk = 1000
    grid = N // blk
    return pl.pallas_call(
        _mid_body,
        grid=(grid,),
        in_specs=[
            pl.BlockSpec((blk, F), lambda i: (i, 0)),
            pl.BlockSpec((4, blk, FQ), lambda i: (0, i, 0)),
            pl.BlockSpec((blk, 1), lambda i: (i, 0)),
            pl.BlockSpec((F, F), lambda i: (0, 0)),
            pl.BlockSpec((F, F), lambda i: (0, 0)),
            pl.BlockSpec((1, F), lambda i: (0, 0)),
        ],
        out_specs=[
            pl.BlockSpec((blk, F), lambda i: (i, 0)),
            pl.BlockSpec((4, blk, FQ), lambda i: (0, i, 0)),
        ],
        out_shape=[
            jax.ShapeDtypeStruct((N, F), jnp.float32),
            jax.ShapeDtypeStruct((4, NPAD, FQ), jnp.float32),
        ],
    )(x, S, dis, W0, W1, b.reshape(1, F))


def _final_body(x_ref, s_ref, dis_ref, w0_ref, w1_ref, b_ref, o_ref):
    ndis = -dis_ref[...]
    t = jnp.concatenate([s_ref[k] for k in range(4)], axis=1) * ndis
    o_ref[...] = (jnp.dot(x_ref[...], w0_ref[...],
                          preferred_element_type=jnp.float32)
                  + jnp.dot(t, w1_ref[...], preferred_element_type=jnp.float32)
                  + b_ref[...])


def _final_call(h, S, dis, W0, W1, b):
    blk = 1000
    grid = N // blk
    return pl.pallas_call(
        _final_body,
        grid=(grid,),
        in_specs=[
            pl.BlockSpec((blk, F), lambda i: (i, 0)),
            pl.BlockSpec((4, blk, FQ), lambda i: (0, i, 0)),
            pl.BlockSpec((blk, 1), lambda i: (i, 0)),
            pl.BlockSpec((F, F), lambda i: (0, 0)),
            pl.BlockSpec((F, F), lambda i: (0, 0)),
            pl.BlockSpec((1, F), lambda i: (0, 0)),
        ],
        out_specs=pl.BlockSpec((blk, F), lambda i: (i, 0)),
        out_shape=jax.ShapeDtypeStruct((N, F), jnp.float32),
    )(h, S, dis, W0, W1, b.reshape(1, F))


def kernel(x, adj, W0a, W1a, ba, W0b, W1b, bb):
    pad = EPAD - E
    padv = jnp.full((pad,), N, jnp.int32)
    srcf = jnp.concatenate([adj[0], padv])
    dstf = jnp.concatenate([adj[1], padv])
    srct = srcf.reshape(NS, NBT, BLK)
    dstt = dstf.reshape(NS, NBT, BLK)
    ones = jnp.ones((BLK,), jnp.float32)
    zer1 = jnp.zeros((ROWS_T,), jnp.float32)
    zrows = jnp.zeros((BLK, FQ), jnp.float32)
    xp = jnp.concatenate([x, jnp.zeros((NPAD - N, F), jnp.float32)])
    x4 = jnp.stack([xp[:, k * FQ:(k + 1) * FQ] for k in range(4)])

    S1, dis2 = _spmm1_kernel(x4, srct, dstt, zrows, ones, zer1)
    dis = dis2[0, :N].reshape(N, 1)
    h, hs4 = _mid_call(x, S1, dis, W0a, W1a, ba)
    S2 = _spmm_kernel(hs4, srct, dstt, zrows)
    return _final_call(h, S2, dis, W0b, W1b, bb)


# speedup vs baseline: 1.0530x; 1.0530x over previous
import functools

import jax
import jax.numpy as jnp
from jax import lax
from jax.experimental import pallas as pl
from jax.experimental.pallas import tpu as pltpu
from jax.experimental.pallas import tpu_sc as plsc

N = 10000
E = 320000
F = 128
NC = 2
NS = 16
NW = NC * NS
BLK = 128
NB = 80
EPAD = NW * NB * BLK
FQ = F // 4
NBT = EPAD // (NS * BLK)
NPAD = 10112
ROWS_T = 640
ROWS_L = NPAD - (NS - 1) * ROWS_T


def _sc_mesh():
    return plsc.VectorSubcoreMesh(core_axis_name="c", subcore_axis_name="s")


_RSQRT_MAGIC = 0x5F3759DF


def _newton_rsqrt(d):
    i = lax.bitcast_convert_type(d, jnp.int32)
    i = _RSQRT_MAGIC - lax.shift_right_arithmetic(i, 1)
    y = lax.bitcast_convert_type(i, jnp.float32)
    h = 0.5 * d
    for _ in range(3):
        y = y * (1.5 - h * y * y)
    return jnp.where(d >= 0.5, y, 0.0)


@functools.partial(
    pl.kernel,
    out_type=[
        jax.ShapeDtypeStruct((4, NPAD, FQ), jnp.float32),
        jax.ShapeDtypeStruct((NC, NPAD), jnp.float32),
    ],
    mesh=_sc_mesh(),
    scratch_types=[
        pltpu.VMEM((NBT, BLK), jnp.int32),
        pltpu.VMEM((NBT, BLK), jnp.int32),
        pltpu.VMEM((BLK, FQ), jnp.float32),
        pltpu.VMEM((BLK, FQ), jnp.float32),
        pltpu.VMEM((BLK, FQ), jnp.float32),
        pltpu.VMEM((BLK,), jnp.float32),
        pltpu.VMEM((ROWS_T,), jnp.float32),
        pltpu.VMEM_SHARED((NPAD, FQ), jnp.float32),
        pltpu.VMEM_SHARED((NPAD, FQ), jnp.float32),
        pltpu.VMEM_SHARED((NPAD,), jnp.float32),
        pltpu.SemaphoreType.DMA,
        pltpu.SemaphoreType.DMA,
        pltpu.SemaphoreType.DMA,
        pltpu.SemaphoreType.DMA,
        pltpu.SemaphoreType.DMA,
    ],
    compiler_params=pltpu.CompilerParams(use_tc_tiling_on_sc=False),
)
def _spmm1_kernel(x4_hbm, src_hbm, dst_hbm, zrows_hbm, ones_hbm, zer1_hbm,
                  out_hbm, dis_hbm,
                  sidx, didx, rows0, rows1, zrows, onesv, disv,
                  tab, acc, dega, sem0, sem1, sem2, sem0s, sem1s):
    c = lax.axis_index("c")
    s = lax.axis_index("s")
    r0 = s * ROWS_T
    nr = jnp.where(s == NS - 1, ROWS_L, ROWS_T)
    nz = nr // BLK
    nd = jnp.where(s == NS - 1, (N - (NS - 1) * ROWS_T) // 80, ROWS_T // 80)

    pltpu.sync_copy(src_hbm.at[s], sidx)
    pltpu.sync_copy(dst_hbm.at[s], didx)
    pltpu.sync_copy(zrows_hbm, zrows)
    pltpu.sync_copy(ones_hbm, onesv)

    pltpu.sync_copy(zer1_hbm, disv)

    @pl.when(s < NS - 1)
    def _():
        pltpu.sync_copy(disv, dega.at[pl.ds(r0, ROWS_T)])

    @pl.when(s == NS - 1)
    def _():
        pltpu.sync_copy(disv.at[pl.ds(0, ROWS_L)],
                        dega.at[pl.ds(r0, ROWS_L)])

    plsc.subcore_barrier()

    def dfire(j, carry):
        pltpu.async_copy(onesv, dega.at[sidx.at[j]], sem2, add=True)
        return carry

    lax.fori_loop(0, NBT, dfire, 0)

    def ddrain(j, carry):
        pltpu.make_async_copy(onesv, dega.at[sidx.at[0]], sem2).wait()
        return carry

    lax.fori_loop(0, NBT, ddrain, 0)
    plsc.subcore_barrier()

    @pl.when(s < NS - 1)
    def _():
        pltpu.sync_copy(dega.at[pl.ds(r0, ROWS_T)], disv)

    @pl.when(s == NS - 1)
    def _():
        pltpu.sync_copy(dega.at[pl.ds(r0, ROWS_L)], disv.at[pl.ds(0, ROWS_L)])

    def disbody(k, carry):
        d = disv[pl.ds(16 * k, 16)]
        disv[pl.ds(16 * k, 16)] = _newton_rsqrt(d)
        return carry

    lax.fori_loop(0, nr // 16, disbody, 0)

    @pl.when(s < NS - 1)
    def _():
        pltpu.async_copy(disv, dis_hbm.at[c, pl.ds(r0, ROWS_T)], sem2)

    @pl.when(s == NS - 1)
    def _():
        pltpu.async_copy(disv.at[pl.ds(0, ROWS_L)],
                         dis_hbm.at[c, pl.ds(r0, ROWS_L)], sem2)

    for q in range(2):
        fq = 2 * c + q

        pltpu.async_copy(x4_hbm.at[fq, pl.ds(r0, BLK)], rows0, sem0)
        for k in range(5):
            buf = rows0 if k % 2 == 0 else rows1
            sem = sem0 if k % 2 == 0 else sem1
            nbuf = rows1 if k % 2 == 0 else rows0
            nsem = sem1 if k % 2 == 0 else sem0

            @pl.when(k < nz)
            def _(k=k, buf=buf, sem=sem, nbuf=nbuf, nsem=nsem):
                pltpu.make_async_copy(x4_hbm.at[fq, pl.ds(r0, BLK)], buf,
                                      sem).wait()

                @pl.when(k + 1 < nz)
                def _():
                    pltpu.async_copy(
                        x4_hbm.at[fq, pl.ds(r0 + (k + 1) * BLK, BLK)],
                        nbuf, nsem)

                def srow(r, carry, k=k, buf=buf):
                    d = disv[pl.ds(k * BLK + r, 1)][0]
                    buf[r, pl.ds(0, 16)] = buf[r, pl.ds(0, 16)] * d
                    buf[r, pl.ds(16, 16)] = buf[r, pl.ds(16, 16)] * d
                    return carry

                lax.fori_loop(0, BLK, srow, 0)
                pltpu.sync_copy(buf, tab.at[pl.ds(r0 + k * BLK, BLK)])
                pltpu.sync_copy(zrows, acc.at[pl.ds(r0 + k * BLK, BLK)])

        plsc.subcore_barrier()

        pltpu.async_copy(tab.at[sidx.at[0]], rows0, sem0)

        def body(i, carry):
            t0 = 2 * i
            pltpu.async_copy(tab.at[sidx.at[t0 + 1]], rows1, sem1)
            pltpu.make_async_copy(tab.at[sidx.at[t0]], rows0, sem0).wait()
            pltpu.sync_copy(rows0, acc.at[didx.at[t0]], add=True)

            @pl.when(t0 + 2 < NBT)
            def _():
                pltpu.async_copy(tab.at[sidx.at[t0 + 2]], rows0, sem0)

            pltpu.make_async_copy(tab.at[sidx.at[t0 + 1]], rows1, sem1).wait()
            pltpu.sync_copy(rows1, acc.at[didx.at[t0 + 1]], add=True)
            return carry

        lax.fori_loop(0, NBT // 2, body, 0)
        plsc.subcore_barrier()

        for k in range(ROWS_T // 80):
            buf = rows0 if k % 2 == 0 else rows1
            sem = sem0 if k % 2 == 0 else sem1

            @pl.when(k < nd)
            def _(k=k, buf=buf, sem=sem):
                if k >= 2:
                    pltpu.make_async_copy(buf.at[pl.ds(0, 80)],
                                          out_hbm.at[fq, pl.ds(r0, 80)],
                                          sem).wait()
                pltpu.sync_copy(acc.at[pl.ds(r0 + k * 80, 80)],
                                buf.at[pl.ds(0, 80)])
                pltpu.async_copy(buf.at[pl.ds(0, 80)],
                                 out_hbm.at[fq, pl.ds(r0 + k * 80, 80)], sem)

        for k in range(2):
            buf = rows0 if k == 0 else rows1
            sem = sem0 if k == 0 else sem1

            @pl.when(k < nd)
            def _(buf=buf, sem=sem):
                pltpu.make_async_copy(buf.at[pl.ds(0, 80)],
                                      out_hbm.at[fq, pl.ds(r0, 80)], sem).wait()

        plsc.subcore_barrier()

    @pl.when(s < NS - 1)
    def _():
        pltpu.make_async_copy(disv, dis_hbm.at[c, pl.ds(r0, ROWS_T)],
                              sem2).wait()

    @pl.when(s == NS - 1)
    def _():
        pltpu.make_async_copy(disv.at[pl.ds(0, ROWS_L)],
                              dis_hbm.at[c, pl.ds(r0, ROWS_L)], sem2).wait()


@functools.partial(
    pl.kernel,
    out_type=jax.ShapeDtypeStruct((4, NPAD, FQ), jnp.float32),
    mesh=_sc_mesh(),
    scratch_types=[
        pltpu.VMEM((NBT, BLK), jnp.int32),
        pltpu.VMEM((NBT, BLK), jnp.int32),
        pltpu.VMEM((BLK, FQ), jnp.float32),
        pltpu.VMEM((BLK, FQ), jnp.float32),
        pltpu.VMEM((BLK, FQ), jnp.float32),
        pltpu.VMEM_SHARED((NPAD, FQ), jnp.float32),
        pltpu.VMEM_SHARED((NPAD, FQ), jnp.float32),
        pltpu.SemaphoreType.DMA,
        pltpu.SemaphoreType.DMA,
        pltpu.SemaphoreType.DMA,
        pltpu.SemaphoreType.DMA,
    ],
    compiler_params=pltpu.CompilerParams(use_tc_tiling_on_sc=False),
)
def _spmm_kernel(tabs_hbm, src_hbm, dst_hbm, zrows_hbm, out_hbm,
                 sidx, didx, rows0, rows1, zrows, tab, acc,
                 sem0, sem1, sem0s, sem1s):
    c = lax.axis_index("c")
    s = lax.axis_index("s")
    r0 = s * ROWS_T
    nr = jnp.where(s == NS - 1, ROWS_L, ROWS_T)
    nz = nr // BLK
    nd = jnp.where(s == NS - 1, (N - (NS - 1) * ROWS_T) // 80, ROWS_T // 80)

    pltpu.sync_copy(src_hbm.at[s], sidx)
    pltpu.sync_copy(dst_hbm.at[s], didx)
    pltpu.sync_copy(zrows_hbm, zrows)

    for q in range(2):
        fq = 2 * c + q

        pltpu.async_copy(tabs_hbm.at[fq, pl.ds(r0, BLK)], rows0, sem0)
        for k in range(5):
            buf = rows0 if k % 2 == 0 else rows1
            sem = sem0 if k % 2 == 0 else sem1
            nbuf = rows1 if k % 2 == 0 else rows0
            nsem = sem1 if k % 2 == 0 else sem0

            @pl.when(k < nz)
            def _(k=k, buf=buf, sem=sem, nbuf=nbuf, nsem=nsem):
                pltpu.make_async_copy(tabs_hbm.at[fq, pl.ds(r0, BLK)], buf,
                                      sem).wait()

                @pl.when(k + 1 < nz)
                def _():
                    pltpu.async_copy(
                        tabs_hbm.at[fq, pl.ds(r0 + (k + 1) * BLK, BLK)],
                        nbuf, nsem)

                pltpu.sync_copy(buf, tab.at[pl.ds(r0 + k * BLK, BLK)])
                pltpu.sync_copy(zrows, acc.at[pl.ds(r0 + k * BLK, BLK)])

        plsc.subcore_barrier()

        pltpu.async_copy(tab.at[sidx.at[0]], rows0, sem0)

        def body(i, carry):
            t0 = 2 * i
            pltpu.async_copy(tab.at[sidx.at[t0 + 1]], rows1, sem1)
            pltpu.make_async_copy(tab.at[sidx.at[t0]], rows0, sem0).wait()
            pltpu.sync_copy(rows0, acc.at[didx.at[t0]], add=True)

            @pl.when(t0 + 2 < NBT)
            def _():
                pltpu.async_copy(tab.at[sidx.at[t0 + 2]], rows0, sem0)

            pltpu.make_async_copy(tab.at[sidx.at[t0 + 1]], rows1, sem1).wait()
            pltpu.sync_copy(rows1, acc.at[didx.at[t0 + 1]], add=True)
            return carry

        lax.fori_loop(0, NBT // 2, body, 0)
        plsc.subcore_barrier()

        for k in range(ROWS_T // 80):
            buf = rows0 if k % 2 == 0 else rows1
            sem = sem0 if k % 2 == 0 else sem1

            @pl.when(k < nd)
            def _(k=k, buf=buf, sem=sem):
                if k >= 2:
                    pltpu.make_async_copy(buf.at[pl.ds(0, 80)],
                                          out_hbm.at[fq, pl.ds(r0, 80)],
                                          sem).wait()
                pltpu.sync_copy(acc.at[pl.ds(r0 + k * 80, 80)],
                                buf.at[pl.ds(0, 80)])
                pltpu.async_copy(buf.at[pl.ds(0, 80)],
                                 out_hbm.at[fq, pl.ds(r0 + k * 80, 80)], sem)

        for k in range(2):
            buf = rows0 if k == 0 else rows1
            sem = sem0 if k == 0 else sem1

            @pl.when(k < nd)
            def _(buf=buf, sem=sem):
                pltpu.make_async_copy(buf.at[pl.ds(0, 80)],
                                      out_hbm.at[fq, pl.ds(r0, 80)], sem).wait()

        plsc.subcore_barrier()


def _mid_body(x_ref, s_ref, dis_ref, w0_ref, w1_ref, b_ref, h_ref, hs4_ref):
    ndis = -dis_ref[...]
    t = jnp.concatenate([s_ref[k] for k in range(4)], axis=1) * ndis
    h = (jnp.dot(x_ref[...], w0_ref[...], preferred_element_type=jnp.float32)
         + jnp.dot(t, w1_ref[...], preferred_element_type=jnp.float32)
         + b_ref[...])
    h = jnp.maximum(h, 0.0)
    h_ref[...] = h
    hs = h * dis_ref[...]
    for k in range(4):
        hs4_ref[k] = hs[:, k * FQ:(k + 1) * FQ]


def _mid_call(x, S, dis, W0, W1, b):
    blk = 1000
    grid = N // blk
    return pl.pallas_call(
        _mid_body,
        grid=(grid,),
        in_specs=[
            pl.BlockSpec((blk, F), lambda i: (i, 0)),
            pl.BlockSpec((4, blk, FQ), lambda i: (0, i, 0)),
            pl.BlockSpec((blk, 1), lambda i: (i, 0)),
            pl.BlockSpec((F, F), lambda i: (0, 0)),
            pl.BlockSpec((F, F), lambda i: (0, 0)),
            pl.BlockSpec((1, F), lambda i: (0, 0)),
        ],
        out_specs=[
            pl.BlockSpec((blk, F), lambda i: (i, 0)),
            pl.BlockSpec((4, blk, FQ), lambda i: (0, i, 0)),
        ],
        out_shape=[
            jax.ShapeDtypeStruct((N, F), jnp.float32),
            jax.ShapeDtypeStruct((4, NPAD, FQ), jnp.float32),
        ],
    )(x, S, dis, W0, W1, b.reshape(1, F))


def _final_body(x_ref, s_ref, dis_ref, w0_ref, w1_ref, b_ref, o_ref):
    ndis = -dis_ref[...]
    t = jnp.concatenate([s_ref[k] for k in range(4)], axis=1) * ndis
    o_ref[...] = (jnp.dot(x_ref[...], w0_ref[...],
                          preferred_element_type=jnp.float32)
                  + jnp.dot(t, w1_ref[...], preferred_element_type=jnp.float32)
                  + b_ref[...])


def _final_call(h, S, dis, W0, W1, b):
    blk = 1000
    grid = N // blk
    return pl.pallas_call(
        _final_body,
        grid=(grid,),
        in_specs=[
            pl.BlockSpec((blk, F), lambda i: (i, 0)),
            pl.BlockSpec((4, blk, FQ), lambda i: (0, i, 0)),
            pl.BlockSpec((blk, 1), lambda i: (i, 0)),
            pl.BlockSpec((F, F), lambda i: (0, 0)),
            pl.BlockSpec((F, F), lambda i: (0, 0)),
            pl.BlockSpec((1, F), lambda i: (0, 0)),
        ],
        out_specs=pl.BlockSpec((blk, F), lambda i: (i, 0)),
        out_shape=jax.ShapeDtypeStruct((N, F), jnp.float32),
    )(h, S, dis, W0, W1, b.reshape(1, F))


def kernel(x, adj, W0a, W1a, ba, W0b, W1b, bb):
    pad = EPAD - E
    padv = jnp.full((pad,), N, jnp.int32)
    srcf = jnp.concatenate([adj[0], padv])
    dstf = jnp.concatenate([adj[1], padv])
    srct = srcf.reshape(NS, NBT, BLK)
    dstt = dstf.reshape(NS, NBT, BLK)
    ones = jnp.ones((BLK,), jnp.float32)
    zer1 = jnp.zeros((ROWS_T,), jnp.float32)
    zrows = jnp.zeros((BLK, FQ), jnp.float32)
    xp = jnp.concatenate([x, jnp.zeros((NPAD - N, F), jnp.float32)])
    x4 = jnp.stack([xp[:, k * FQ:(k + 1) * FQ] for k in range(4)])

    S1, dis2 = _spmm1_kernel(x4, srct, dstt, zrows, ones, zer1)
    dis = dis2[0, :N].reshape(N, 1)
    h, hs4 = _mid_call(x, S1, dis, W0a, W1a, ba)
    S2 = _spmm_kernel(hs4, srct, dstt, zrows)
    return _final_call(h, S2, dis, W0b, W1b, bb)


# speedup vs baseline: 1.0566x; 1.0034x over previous
import functools

import jax
import jax.numpy as jnp
from jax import lax
from jax.experimental import pallas as pl
from jax.experimental.pallas import tpu as pltpu
from jax.experimental.pallas import tpu_sc as plsc

N = 10000
E = 320000
F = 128
NC = 2
NS = 16
NW = NC * NS
BLK = 128
NB = 80
EPAD = NW * NB * BLK
FQ = F // 4
NBT = EPAD // (NS * BLK)
NPAD = 10112
ROWS_T = 640
ROWS_L = NPAD - (NS - 1) * ROWS_T


def _sc_mesh():
    return plsc.VectorSubcoreMesh(core_axis_name="c", subcore_axis_name="s")


_RSQRT_MAGIC = 0x5F3759DF


def _newton_rsqrt(d):
    i = lax.bitcast_convert_type(d, jnp.int32)
    i = _RSQRT_MAGIC - lax.shift_right_arithmetic(i, 1)
    y = lax.bitcast_convert_type(i, jnp.float32)
    h = 0.5 * d
    for _ in range(3):
        y = y * (1.5 - h * y * y)
    return jnp.where(d >= 0.5, y, 0.0)


@functools.partial(
    pl.kernel,
    out_type=[
        jax.ShapeDtypeStruct((4, NPAD, FQ), jnp.float32),
        jax.ShapeDtypeStruct((NC, NPAD), jnp.float32),
    ],
    mesh=_sc_mesh(),
    scratch_types=[
        pltpu.VMEM((NBT, BLK), jnp.int32),
        pltpu.VMEM((NBT, BLK), jnp.int32),
        pltpu.VMEM((BLK, FQ), jnp.float32),
        pltpu.VMEM((BLK, FQ), jnp.float32),
        pltpu.VMEM((BLK, FQ), jnp.float32),
        pltpu.VMEM((BLK,), jnp.float32),
        pltpu.VMEM((ROWS_T,), jnp.float32),
        pltpu.VMEM((ROWS_T, FQ), jnp.float32),
        pltpu.VMEM_SHARED((NPAD, FQ), jnp.float32),
        pltpu.VMEM_SHARED((NPAD, FQ), jnp.float32),
        pltpu.VMEM_SHARED((NPAD,), jnp.float32),
        pltpu.SemaphoreType.DMA,
        pltpu.SemaphoreType.DMA,
        pltpu.SemaphoreType.DMA,
        pltpu.SemaphoreType.DMA,
        pltpu.SemaphoreType.DMA,
    ],
    compiler_params=pltpu.CompilerParams(use_tc_tiling_on_sc=False),
)
def _spmm1_kernel(x4_hbm, src_hbm, dst_hbm, zrows_hbm, ones_hbm, zer1_hbm,
                  out_hbm, dis_hbm,
                  sidx, didx, rows0, rows1, zrows, onesv, disv, xstg,
                  tab, acc, dega, sem0, sem1, sem2, sem0s, sem1s):
    c = lax.axis_index("c")
    s = lax.axis_index("s")
    r0 = s * ROWS_T
    nr = jnp.where(s == NS - 1, ROWS_L, ROWS_T)
    nz = nr // BLK
    nd = jnp.where(s == NS - 1, (N - (NS - 1) * ROWS_T) // 80, ROWS_T // 80)

    pltpu.sync_copy(src_hbm.at[s], sidx)
    pltpu.sync_copy(dst_hbm.at[s], didx)
    pltpu.sync_copy(zrows_hbm, zrows)
    pltpu.sync_copy(ones_hbm, onesv)

    pltpu.sync_copy(zer1_hbm, disv)

    @pl.when(s < NS - 1)
    def _():
        pltpu.sync_copy(disv, dega.at[pl.ds(r0, ROWS_T)])

    @pl.when(s == NS - 1)
    def _():
        pltpu.sync_copy(disv.at[pl.ds(0, ROWS_L)],
                        dega.at[pl.ds(r0, ROWS_L)])

    plsc.subcore_barrier()

    def dfire(j, carry):
        pltpu.async_copy(onesv, dega.at[sidx.at[j]], sem2, add=True)
        return carry

    lax.fori_loop(0, NBT, dfire, 0)

    @pl.when(s < NS - 1)
    def _():
        pltpu.async_copy(x4_hbm.at[2 * c, pl.ds(r0, ROWS_T)], xstg, sem0s)

    @pl.when(s == NS - 1)
    def _():
        pltpu.async_copy(x4_hbm.at[2 * c, pl.ds(r0, ROWS_L)],
                         xstg.at[pl.ds(0, ROWS_L)], sem0s)

    for k in range(5):
        @pl.when(k < nz)
        def _(k=k):
            pltpu.sync_copy(zrows, acc.at[pl.ds(r0 + k * BLK, BLK)])

    def ddrain(j, carry):
        pltpu.make_async_copy(onesv, dega.at[sidx.at[0]], sem2).wait()
        return carry

    lax.fori_loop(0, NBT, ddrain, 0)
    plsc.subcore_barrier()

    @pl.when(s < NS - 1)
    def _():
        pltpu.sync_copy(dega.at[pl.ds(r0, ROWS_T)], disv)

    @pl.when(s == NS - 1)
    def _():
        pltpu.sync_copy(dega.at[pl.ds(r0, ROWS_L)], disv.at[pl.ds(0, ROWS_L)])

    def disbody(k, carry):
        d = disv[pl.ds(16 * k, 16)]
        disv[pl.ds(16 * k, 16)] = _newton_rsqrt(d)
        return carry

    lax.fori_loop(0, nr // 16, disbody, 0)

    @pl.when(s < NS - 1)
    def _():
        pltpu.async_copy(disv, dis_hbm.at[c, pl.ds(r0, ROWS_T)], sem2)

    @pl.when(s == NS - 1)
    def _():
        pltpu.async_copy(disv.at[pl.ds(0, ROWS_L)],
                         dis_hbm.at[c, pl.ds(r0, ROWS_L)], sem2)

    for q in range(2):
        fq = 2 * c + q
        stgsem = sem0s if q == 0 else sem1s

        if q == 1:
            for k in range(5):
                @pl.when(k < nz)
                def _(k=k):
                    pltpu.sync_copy(zrows, acc.at[pl.ds(r0 + k * BLK, BLK)])

        @pl.when(s < NS - 1)
        def _(xstg=xstg, stgsem=stgsem):
            pltpu.make_async_copy(x4_hbm.at[fq, pl.ds(r0, ROWS_T)], xstg,
                                  stgsem).wait()

        @pl.when(s == NS - 1)
        def _(xstg=xstg, stgsem=stgsem):
            pltpu.make_async_copy(x4_hbm.at[fq, pl.ds(r0, ROWS_L)],
                                  xstg.at[pl.ds(0, ROWS_L)], stgsem).wait()

        def srow(r, carry, xstg=xstg):
            d = disv[pl.ds(r, 1)][0]
            xstg[r, pl.ds(0, 16)] = xstg[r, pl.ds(0, 16)] * d
            xstg[r, pl.ds(16, 16)] = xstg[r, pl.ds(16, 16)] * d
            return carry

        lax.fori_loop(0, nr, srow, 0)

        @pl.when(s < NS - 1)
        def _(xstg=xstg):
            pltpu.sync_copy(xstg, tab.at[pl.ds(r0, ROWS_T)])

        @pl.when(s == NS - 1)
        def _(xstg=xstg):
            pltpu.sync_copy(xstg.at[pl.ds(0, ROWS_L)],
                            tab.at[pl.ds(r0, ROWS_L)])

        if q == 0:
            @pl.when(s < NS - 1)
            def _():
                pltpu.async_copy(x4_hbm.at[2 * c + 1, pl.ds(r0, ROWS_T)],
                                 xstg, sem1s)

            @pl.when(s == NS - 1)
            def _():
                pltpu.async_copy(x4_hbm.at[2 * c + 1, pl.ds(r0, ROWS_L)],
                                 xstg.at[pl.ds(0, ROWS_L)], sem1s)

        plsc.subcore_barrier()

        pltpu.async_copy(tab.at[sidx.at[0]], rows0, sem0)

        def body(i, carry):
            t0 = 2 * i
            pltpu.async_copy(tab.at[sidx.at[t0 + 1]], rows1, sem1)
            pltpu.make_async_copy(tab.at[sidx.at[t0]], rows0, sem0).wait()
            pltpu.sync_copy(rows0, acc.at[didx.at[t0]], add=True)

            @pl.when(t0 + 2 < NBT)
            def _():
                pltpu.async_copy(tab.at[sidx.at[t0 + 2]], rows0, sem0)

            pltpu.make_async_copy(tab.at[sidx.at[t0 + 1]], rows1, sem1).wait()
            pltpu.sync_copy(rows1, acc.at[didx.at[t0 + 1]], add=True)
            return carry

        lax.fori_loop(0, NBT // 2, body, 0)
        plsc.subcore_barrier()

        for k in range(ROWS_T // 80):
            buf = rows0 if k % 2 == 0 else rows1
            sem = sem0 if k % 2 == 0 else sem1

            @pl.when(k < nd)
            def _(k=k, buf=buf, sem=sem):
                if k >= 2:
                    pltpu.make_async_copy(buf.at[pl.ds(0, 80)],
                                          out_hbm.at[fq, pl.ds(r0, 80)],
                                          sem).wait()
                pltpu.sync_copy(acc.at[pl.ds(r0 + k * 80, 80)],
                                buf.at[pl.ds(0, 80)])
                pltpu.async_copy(buf.at[pl.ds(0, 80)],
                                 out_hbm.at[fq, pl.ds(r0 + k * 80, 80)], sem)

        for k in range(2):
            buf = rows0 if k == 0 else rows1
            sem = sem0 if k == 0 else sem1

            @pl.when(k < nd)
            def _(buf=buf, sem=sem):
                pltpu.make_async_copy(buf.at[pl.ds(0, 80)],
                                      out_hbm.at[fq, pl.ds(r0, 80)], sem).wait()

    @pl.when(s < NS - 1)
    def _():
        pltpu.make_async_copy(disv, dis_hbm.at[c, pl.ds(r0, ROWS_T)],
                              sem2).wait()

    @pl.when(s == NS - 1)
    def _():
        pltpu.make_async_copy(disv.at[pl.ds(0, ROWS_L)],
                              dis_hbm.at[c, pl.ds(r0, ROWS_L)], sem2).wait()


@functools.partial(
    pl.kernel,
    out_type=jax.ShapeDtypeStruct((4, NPAD, FQ), jnp.float32),
    mesh=_sc_mesh(),
    scratch_types=[
        pltpu.VMEM((NBT, BLK), jnp.int32),
        pltpu.VMEM((NBT, BLK), jnp.int32),
        pltpu.VMEM((BLK, FQ), jnp.float32),
        pltpu.VMEM((BLK, FQ), jnp.float32),
        pltpu.VMEM((BLK, FQ), jnp.float32),
        pltpu.VMEM_SHARED((NPAD, FQ), jnp.float32),
        pltpu.VMEM_SHARED((NPAD, FQ), jnp.float32),
        pltpu.SemaphoreType.DMA,
        pltpu.SemaphoreType.DMA,
        pltpu.SemaphoreType.DMA,
        pltpu.SemaphoreType.DMA,
    ],
    compiler_params=pltpu.CompilerParams(use_tc_tiling_on_sc=False),
)
def _spmm_kernel(tabs_hbm, src_hbm, dst_hbm, zrows_hbm, out_hbm,
                 sidx, didx, rows0, rows1, zrows, tab, acc,
                 sem0, sem1, sem0s, sem1s):
    c = lax.axis_index("c")
    s = lax.axis_index("s")
    r0 = s * ROWS_T
    nr = jnp.where(s == NS - 1, ROWS_L, ROWS_T)
    nz = nr // BLK
    nd = jnp.where(s == NS - 1, (N - (NS - 1) * ROWS_T) // 80, ROWS_T // 80)

    pltpu.sync_copy(src_hbm.at[s], sidx)
    pltpu.sync_copy(dst_hbm.at[s], didx)
    pltpu.sync_copy(zrows_hbm, zrows)

    for q in range(2):
        fq = 2 * c + q

        pltpu.async_copy(tabs_hbm.at[fq, pl.ds(r0, BLK)], rows0, sem0)
        for k in range(5):
            buf = rows0 if k % 2 == 0 else rows1
            sem = sem0 if k % 2 == 0 else sem1
            nbuf = rows1 if k % 2 == 0 else rows0
            nsem = sem1 if k % 2 == 0 else sem0

            @pl.when(k < nz)
            def _(k=k, buf=buf, sem=sem, nbuf=nbuf, nsem=nsem):
                pltpu.make_async_copy(tabs_hbm.at[fq, pl.ds(r0, BLK)], buf,
                                      sem).wait()

                @pl.when(k + 1 < nz)
                def _():
                    pltpu.async_copy(
                        tabs_hbm.at[fq, pl.ds(r0 + (k + 1) * BLK, BLK)],
                        nbuf, nsem)

                pltpu.sync_copy(buf, tab.at[pl.ds(r0 + k * BLK, BLK)])
                pltpu.sync_copy(zrows, acc.at[pl.ds(r0 + k * BLK, BLK)])

        plsc.subcore_barrier()

        pltpu.async_copy(tab.at[sidx.at[0]], rows0, sem0)

        def body(i, carry):
            t0 = 2 * i
            pltpu.async_copy(tab.at[sidx.at[t0 + 1]], rows1, sem1)
            pltpu.make_async_copy(tab.at[sidx.at[t0]], rows0, sem0).wait()
            pltpu.sync_copy(rows0, acc.at[didx.at[t0]], add=True)

            @pl.when(t0 + 2 < NBT)
            def _():
                pltpu.async_copy(tab.at[sidx.at[t0 + 2]], rows0, sem0)

            pltpu.make_async_copy(tab.at[sidx.at[t0 + 1]], rows1, sem1).wait()
            pltpu.sync_copy(rows1, acc.at[didx.at[t0 + 1]], add=True)
            return carry

        lax.fori_loop(0, NBT // 2, body, 0)
        plsc.subcore_barrier()

        for k in range(ROWS_T // 80):
            buf = rows0 if k % 2 == 0 else rows1
            sem = sem0 if k % 2 == 0 else sem1

            @pl.when(k < nd)
            def _(k=k, buf=buf, sem=sem):
                if k >= 2:
                    pltpu.make_async_copy(buf.at[pl.ds(0, 80)],
                                          out_hbm.at[fq, pl.ds(r0, 80)],
                                          sem).wait()
                pltpu.sync_copy(acc.at[pl.ds(r0 + k * 80, 80)],
                                buf.at[pl.ds(0, 80)])
                pltpu.async_copy(buf.at[pl.ds(0, 80)],
                                 out_hbm.at[fq, pl.ds(r0 + k * 80, 80)], sem)

        for k in range(2):
            buf = rows0 if k == 0 else rows1
            sem = sem0 if k == 0 else sem1

            @pl.when(k < nd)
            def _(buf=buf, sem=sem):
                pltpu.make_async_copy(buf.at[pl.ds(0, 80)],
                                      out_hbm.at[fq, pl.ds(r0, 80)], sem).wait()

        plsc.subcore_barrier()


def _mid_body(x_ref, s_ref, dis_ref, w0_ref, w1_ref, b_ref, h_ref, hs4_ref):
    ndis = -dis_ref[...]
    t = jnp.concatenate([s_ref[k] for k in range(4)], axis=1) * ndis
    h = (jnp.dot(x_ref[...], w0_ref[...], preferred_element_type=jnp.float32)
         + jnp.dot(t, w1_ref[...], preferred_element_type=jnp.float32)
         + b_ref[...])
    h = jnp.maximum(h, 0.0)
    h_ref[...] = h
    hs = h * dis_ref[...]
    for k in range(4):
        hs4_ref[k] = hs[:, k * FQ:(k + 1) * FQ]


def _mid_call(x, S, dis, W0, W1, b):
    blk = 1000
    grid = N // blk
    return pl.pallas_call(
        _mid_body,
        grid=(grid,),
        in_specs=[
            pl.BlockSpec((blk, F), lambda i: (i, 0)),
            pl.BlockSpec((4, blk, FQ), lambda i: (0, i, 0)),
            pl.BlockSpec((blk, 1), lambda i: (i, 0)),
            pl.BlockSpec((F, F), lambda i: (0, 0)),
            pl.BlockSpec((F, F), lambda i: (0, 0)),
            pl.BlockSpec((1, F), lambda i: (0, 0)),
        ],
        out_specs=[
            pl.BlockSpec((blk, F), lambda i: (i, 0)),
            pl.BlockSpec((4, blk, FQ), lambda i: (0, i, 0)),
        ],
        out_shape=[
            jax.ShapeDtypeStruct((N, F), jnp.float32),
            jax.ShapeDtypeStruct((4, NPAD, FQ), jnp.float32),
        ],
    )(x, S, dis, W0, W1, b.reshape(1, F))


def _final_body(x_ref, s_ref, dis_ref, w0_ref, w1_ref, b_ref, o_ref):
    ndis = -dis_ref[...]
    t = jnp.concatenate([s_ref[k] for k in range(4)], axis=1) * ndis
    o_ref[...] = (jnp.dot(x_ref[...], w0_ref[...],
                          preferred_element_type=jnp.float32)
                  + jnp.dot(t, w1_ref[...], preferred_element_type=jnp.float32)
                  + b_ref[...])


def _final_call(h, S, dis, W0, W1, b):
    blk = 1000
    grid = N // blk
    return pl.pallas_call(
        _final_body,
        grid=(grid,),
        in_specs=[
            pl.BlockSpec((blk, F), lambda i: (i, 0)),
            pl.BlockSpec((4, blk, FQ), lambda i: (0, i, 0)),
            pl.BlockSpec((blk, 1), lambda i: (i, 0)),
            pl.BlockSpec((F, F), lambda i: (0, 0)),
            pl.BlockSpec((F, F), lambda i: (0, 0)),
            pl.BlockSpec((1, F), lambda i: (0, 0)),
        ],
        out_specs=pl.BlockSpec((blk, F), lambda i: (i, 0)),
        out_shape=jax.ShapeDtypeStruct((N, F), jnp.float32),
    )(h, S, dis, W0, W1, b.reshape(1, F))


def kernel(x, adj, W0a, W1a, ba, W0b, W1b, bb):
    pad = EPAD - E
    padv = jnp.full((pad,), N, jnp.int32)
    srcf = jnp.concatenate([adj[0], padv])
    dstf = jnp.concatenate([adj[1], padv])
    srct = srcf.reshape(NS, NBT, BLK)
    dstt = dstf.reshape(NS, NBT, BLK)
    ones = jnp.ones((BLK,), jnp.float32)
    zer1 = jnp.zeros((ROWS_T,), jnp.float32)
    zrows = jnp.zeros((BLK, FQ), jnp.float32)
    xp = jnp.concatenate([x, jnp.zeros((NPAD - N, F), jnp.float32)])
    x4 = jnp.stack([xp[:, k * FQ:(k + 1) * FQ] for k in range(4)])

    S1, dis2 = _spmm1_kernel(x4, srct, dstt, zrows, ones, zer1)
    dis = dis2[0, :N].reshape(N, 1)
    h, hs4 = _mid_call(x, S1, dis, W0a, W1a, ba)
    S2 = _spmm_kernel(hs4, srct, dstt, zrows)
    return _final_call(h, S2, dis, W0b, W1b, bb)
